# baseline (speedup 1.0000x reference)
import functools

import jax
import jax.numpy as jnp
from jax import lax
from jax.experimental import pallas as pl
from jax.experimental.pallas import tpu as pltpu
from jax.experimental.pallas import tpu_sc as plsc

B = 1024
D = 128
VOCAB = 100000
LANES = 16
NC = 2
NS = 16
NW = NC * NS
BPW = B // NW


def _sc_gather_dot(i32, j32, v_w, w_w, bcat):
    mesh = plsc.VectorSubcoreMesh(
        core_axis_name="c", subcore_axis_name="s", num_cores=NC, num_subcores=NS
    )

    @functools.partial(
        pl.kernel,
        mesh=mesh,
        compiler_params=pltpu.CompilerParams(needs_layout_passes=False),
        out_type=[
            jax.ShapeDtypeStruct((B, LANES), jnp.float32),
            jax.ShapeDtypeStruct((B,), jnp.float32),
        ],
        scratch_types=[
            pltpu.VMEM((BPW,), jnp.int32),
            pltpu.VMEM((BPW,), jnp.int32),
            pltpu.VMEM((BPW,), jnp.int32),
            pltpu.VMEM((BPW,), jnp.int32),
            pltpu.VMEM((BPW, D), jnp.float32),
            pltpu.VMEM((BPW, D), jnp.float32),
            pltpu.VMEM((BPW, D), jnp.float32),
            pltpu.VMEM((BPW, D), jnp.float32),
            pltpu.VMEM((BPW, LANES), jnp.float32),
            pltpu.VMEM((BPW,), jnp.float32),
            pltpu.SemaphoreType.DMA,
            pltpu.SemaphoreType.DMA,
            pltpu.SemaphoreType.DMA,
        ],
    )
    def sc_k(i_hbm, j_hbm, v_hbm, w_hbm, b_hbm,
             simp_hbm, bsum_hbm,
             iv, jv, ivh, jvh, vrows, wrows, bvrows, bwrows, simp, bsumv,
             sem, bsem, isem):
        wid = lax.axis_index("s") * NC + lax.axis_index("c")
        base = wid * BPW
        icp = pltpu.async_copy(i_hbm.at[pl.ds(base, BPW)], iv, isem)
        jcp = pltpu.async_copy(j_hbm.at[pl.ds(base, BPW)], jv, isem)
        icp.wait()
        jcp.wait()
        cps = [
            pltpu.async_copy(v_hbm.at[iv], vrows, sem),
            pltpu.async_copy(w_hbm.at[jv], wrows, sem),
        ]
        for g in range(BPW // LANES):
            sl = pl.ds(g * LANES, LANES)
            ivh[sl] = iv[sl] >> 7
            jvh[sl] = (jv[sl] + VOCAB) >> 7
        bcps = [
            pltpu.async_copy(b_hbm.at[ivh], bvrows, bsem),
            pltpu.async_copy(b_hbm.at[jvh], bwrows, bsem),
        ]
        for cp in cps:
            cp.wait()
        @plsc.parallel_loop(0, BPW, 1, unroll=8)
        def _dot_body(e):
            nk = D // LANES
            acc0 = vrows[e, 0:LANES] * wrows[e, 0:LANES]
            ck = pl.ds(LANES, LANES)
            acc1 = vrows[e, ck] * wrows[e, ck]
            for k in range(2, nk, 2):
                c0 = pl.ds(k * LANES, LANES)
                c1 = pl.ds((k + 1) * LANES, LANES)
                acc0 = acc0 + vrows[e, c0] * wrows[e, c0]
                acc1 = acc1 + vrows[e, c1] * wrows[e, c1]
            simp[e, :] = acc0 + acc1
        for cp in bcps:
            cp.wait()
        lanes = lax.iota(jnp.int32, LANES)
        for g in range(BPW // LANES):
            sl = pl.ds(g * LANES, LANES)
            rows = lanes + g * LANES
            bsumv[sl] = (plsc.load_gather(bvrows, [rows, iv[sl] & 127])
                         + plsc.load_gather(bwrows, [rows, (jv[sl] + VOCAB) & 127]))
        pltpu.sync_copy(simp, simp_hbm.at[pl.ds(base, BPW)])
        pltpu.sync_copy(bsumv, bsum_hbm.at[pl.ds(base, BPW)])

    return sc_k(i32, j32, v_w, w_w, bcat)


def _tc_finish_body(simp_ref, bsum_ref, co_ref, w_ref, out_ref):
    sim = jnp.sum(simp_ref[...], axis=1).reshape(8, B // 8)
    b = bsum_ref[...]
    mb = jnp.sum(b) * (1.0 / B)
    d = b - mb
    varb = jnp.sum(d * d)
    a = sim - jnp.log(co_ref[...]) + mb
    wv = w_ref[...]
    out_ref[0, 0] = 0.5 * (B * jnp.sum(wv * a * a) + varb * jnp.sum(wv))


def _tc_finish(simp, bsum, co, w):
    out = pl.pallas_call(
        _tc_finish_body,
        out_shape=jax.ShapeDtypeStruct((1, 1), jnp.float32),
        out_specs=pl.BlockSpec(memory_space=pltpu.SMEM),
    )(simp, bsum.reshape(8, B // 8), co.reshape(8, B // 8),
      w.reshape(8, B // 8))
    return out[0, 0]


def _cat_bias(bv, bw):
    flat = jnp.concatenate([bv, bw], axis=0).reshape(-1)
    pad = (-flat.shape[0]) % D
    return jnp.pad(flat, (0, pad)).reshape(-1, D)


def kernel(i, j, co_occur, weight, v_weight, w_weight, biasv_weight, biasw_weight):
    simp, bsum = _sc_gather_dot(
        i.astype(jnp.int32), j.astype(jnp.int32),
        v_weight, w_weight, _cat_bias(biasv_weight, biasw_weight))
    return _tc_finish(simp, bsum, co_occur, weight)


# speedup vs baseline: 1.2068x; 1.2068x over previous
import functools

import jax
import jax.numpy as jnp
from jax import lax
from jax.experimental import pallas as pl
from jax.experimental.pallas import tpu as pltpu
from jax.experimental.pallas import tpu_sc as plsc

B = 1024
D = 128
VOCAB = 100000
LANES = 16
NC = 2
NS = 16
NW = NC * NS
BPW = B // NW


def _sc_gather_dot(i32, j32, v_w, w_w, bvp, bwp):
    mesh = plsc.VectorSubcoreMesh(
        core_axis_name="c", subcore_axis_name="s", num_cores=NC, num_subcores=NS
    )

    @functools.partial(
        pl.kernel,
        mesh=mesh,
        compiler_params=pltpu.CompilerParams(needs_layout_passes=False),
        out_type=[
            jax.ShapeDtypeStruct((B, LANES), jnp.float32),
            jax.ShapeDtypeStruct((B,), jnp.float32),
        ],
        scratch_types=[
            pltpu.VMEM((BPW,), jnp.int32),
            pltpu.VMEM((BPW,), jnp.int32),
            pltpu.VMEM((BPW,), jnp.int32),
            pltpu.VMEM((BPW,), jnp.int32),
            pltpu.VMEM((BPW, D), jnp.float32),
            pltpu.VMEM((BPW, D), jnp.float32),
            pltpu.VMEM((BPW, D), jnp.float32),
            pltpu.VMEM((BPW, D), jnp.float32),
            pltpu.VMEM((BPW, LANES), jnp.float32),
            pltpu.VMEM((BPW,), jnp.float32),
            pltpu.SemaphoreType.DMA,
            pltpu.SemaphoreType.DMA,
            pltpu.SemaphoreType.DMA,
        ],
    )
    def sc_k(i_hbm, j_hbm, v_hbm, w_hbm, bv_hbm, bw_hbm,
             simp_hbm, bsum_hbm,
             iv, jv, ivh, jvh, vrows, wrows, bvrows, bwrows, simp, bsumv,
             sem, bsem, isem):
        wid = lax.axis_index("s") * NC + lax.axis_index("c")
        base = wid * BPW
        icp = pltpu.async_copy(i_hbm.at[pl.ds(base, BPW)], iv, isem)
        jcp = pltpu.async_copy(j_hbm.at[pl.ds(base, BPW)], jv, isem)
        icp.wait()
        jcp.wait()
        cps = [
            pltpu.async_copy(v_hbm.at[iv], vrows, sem),
            pltpu.async_copy(w_hbm.at[jv], wrows, sem),
        ]
        for g in range(BPW // LANES):
            sl = pl.ds(g * LANES, LANES)
            ivh[sl] = iv[sl] >> 7
            jvh[sl] = jv[sl] >> 7
        bcps = [
            pltpu.async_copy(bv_hbm.at[ivh], bvrows, bsem),
            pltpu.async_copy(bw_hbm.at[jvh], bwrows, bsem),
        ]
        for cp in cps:
            cp.wait()
        @plsc.parallel_loop(0, BPW, 1, unroll=8)
        def _dot_body(e):
            nk = D // LANES
            acc0 = vrows[e, 0:LANES] * wrows[e, 0:LANES]
            ck = pl.ds(LANES, LANES)
            acc1 = vrows[e, ck] * wrows[e, ck]
            for k in range(2, nk, 2):
                c0 = pl.ds(k * LANES, LANES)
                c1 = pl.ds((k + 1) * LANES, LANES)
                acc0 = acc0 + vrows[e, c0] * wrows[e, c0]
                acc1 = acc1 + vrows[e, c1] * wrows[e, c1]
            simp[e, :] = acc0 + acc1
        for cp in bcps:
            cp.wait()
        lanes = lax.iota(jnp.int32, LANES)
        for g in range(BPW // LANES):
            sl = pl.ds(g * LANES, LANES)
            rows = lanes + g * LANES
            bsumv[sl] = (plsc.load_gather(bvrows, [rows, iv[sl] & 127])
                         + plsc.load_gather(bwrows, [rows, jv[sl] & 127]))
        pltpu.sync_copy(simp, simp_hbm.at[pl.ds(base, BPW)])
        pltpu.sync_copy(bsumv, bsum_hbm.at[pl.ds(base, BPW)])

    return sc_k(i32, j32, v_w, w_w, bvp, bwp)


def _tc_finish_body(simp_ref, bsum_ref, co_ref, w_ref, out_ref):
    sim = jnp.sum(simp_ref[...], axis=1).reshape(8, B // 8)
    b = bsum_ref[...]
    mb = jnp.sum(b) * (1.0 / B)
    d = b - mb
    varb = jnp.sum(d * d)
    a = sim - jnp.log(co_ref[...]) + mb
    wv = w_ref[...]
    out_ref[0, 0] = 0.5 * (B * jnp.sum(wv * a * a) + varb * jnp.sum(wv))


def _tc_finish(simp, bsum, co, w):
    out = pl.pallas_call(
        _tc_finish_body,
        out_shape=jax.ShapeDtypeStruct((1, 1), jnp.float32),
        out_specs=pl.BlockSpec(memory_space=pltpu.SMEM),
    )(simp, bsum.reshape(8, B // 8), co.reshape(8, B // 8),
      w.reshape(8, B // 8))
    return out[0, 0]


def _pad_bias(bias):
    flat = bias.reshape(-1)
    pad = (-flat.shape[0]) % D
    return jnp.pad(flat, (0, pad)).reshape(-1, D)


def kernel(i, j, co_occur, weight, v_weight, w_weight, biasv_weight, biasw_weight):
    simp, bsum = _sc_gather_dot(
        i.astype(jnp.int32), j.astype(jnp.int32),
        v_weight, w_weight, _pad_bias(biasv_weight), _pad_bias(biasw_weight))
    return _tc_finish(simp, bsum, co_occur, weight)
